# baseline (speedup 1.0000x reference)
import jax
import jax.numpy as jnp
from jax import lax
from jax.experimental import pallas as pl
from jax.experimental.pallas import tpu as pltpu
from jax.experimental.pallas import tpu_sc as plsc

NC = 2
NS = 16
NW = NC * NS
B = 128
DEGW = 16


def _l1_body(x_ref, w_ref, y1e_ref, r1_ref):
    yr = jnp.dot(x_ref[...], w_ref[...], preferred_element_type=jnp.float32)
    dh = w_ref.shape[1] // 2
    y1e_ref[:, :dh] = yr[:, :dh]
    y1e_ref[:, dh:] = jnp.ones((yr.shape[0], DEGW), jnp.float32)
    r1_ref[...] = yr[:, dh:]


def _tc_l1(x, w, n_acc, blk):
    d = x.shape[1]
    dh = w.shape[1] // 2
    grid = (n_acc // blk,)
    return pl.pallas_call(
        _l1_body,
        grid=grid,
        in_specs=[pl.BlockSpec((blk, d), lambda i: (i, 0)),
                  pl.BlockSpec((d, 2 * dh), lambda i: (0, 0))],
        out_specs=[pl.BlockSpec((blk, dh + DEGW), lambda i: (i, 0)),
                   pl.BlockSpec((blk, dh), lambda i: (i, 0))],
        out_shape=[jax.ShapeDtypeStruct((n_acc, dh + DEGW), jnp.float32),
                   jax.ShapeDtypeStruct((n_acc, dh), jnp.float32)],
    )(x, w)


def _mid_body(p0_ref, p1_ref, r1_ref, b1_ref, w_ref, y2_ref, r2_ref, rd_ref):
    dh = r1_ref.shape[1]
    deg = p0_ref[:, dh:dh + 1] + p1_ref[:, dh:dh + 1]
    rdeg = 1.0 / jnp.maximum(deg, 1.0)
    h = (p0_ref[:, :dh] + p1_ref[:, :dh]) * rdeg + b1_ref[...] + r1_ref[...]
    h = jnp.maximum(h, 0.0)
    yr = jnp.dot(h, w_ref[...], preferred_element_type=jnp.float32)
    do = w_ref.shape[1] // 2
    y2_ref[...] = yr[:, :do]
    r2_ref[...] = yr[:, do:]
    rd_ref[...] = jnp.broadcast_to(rdeg, (rdeg.shape[0], DEGW))


def _tc_mid(pflat, r1, b1, w, n_acc, blk):
    dh = r1.shape[1]
    do2 = w.shape[1]
    do = do2 // 2
    nblk = n_acc // blk
    grid = (nblk,)
    return pl.pallas_call(
        _mid_body,
        grid=grid,
        in_specs=[pl.BlockSpec((blk, dh + DEGW), lambda i: (i, 0)),
                  pl.BlockSpec((blk, dh + DEGW), lambda i: (i + nblk, 0)),
                  pl.BlockSpec((blk, dh), lambda i: (i, 0)),
                  pl.BlockSpec((1, dh), lambda i: (0, 0)),
                  pl.BlockSpec((dh, do2), lambda i: (0, 0))],
        out_specs=[pl.BlockSpec((blk, do), lambda i: (i, 0)),
                   pl.BlockSpec((blk, do), lambda i: (i, 0)),
                   pl.BlockSpec((blk, DEGW), lambda i: (i, 0))],
        out_shape=[jax.ShapeDtypeStruct((n_acc, do), jnp.float32),
                   jax.ShapeDtypeStruct((n_acc, do), jnp.float32),
                   jax.ShapeDtypeStruct((n_acc, DEGW), jnp.float32)],
    )(pflat, pflat, r1, b1, w)


def _fin_body(q0_ref, q1_ref, rd_ref, r2_ref, b2_ref, o_ref):
    rdeg = rd_ref[:, :1]
    o_ref[...] = (q0_ref[...] + q1_ref[...]) * rdeg + b2_ref[...] + r2_ref[...]


def _tc_fin(qflat, rd, r2, b2, n, n_acc, blk):
    do = r2.shape[1]
    nblk = n_acc // blk
    grid = (nblk,)
    return pl.pallas_call(
        _fin_body,
        grid=grid,
        in_specs=[pl.BlockSpec((blk, do), lambda i: (i, 0)),
                  pl.BlockSpec((blk, do), lambda i: (i + nblk, 0)),
                  pl.BlockSpec((blk, DEGW), lambda i: (i, 0)),
                  pl.BlockSpec((blk, do), lambda i: (i, 0)),
                  pl.BlockSpec((1, do), lambda i: (0, 0))],
        out_specs=pl.BlockSpec((blk, do), lambda i: (i, 0)),
        out_shape=jax.ShapeDtypeStruct((n, do), jnp.float32),
    )(qflat, qflat, rd, r2, b2)


def _sc_aggregate(table, srcr, dstr, zeros_d, n_acc):
    d = table.shape[1]
    nbs = srcr.shape[1]
    nl = (nbs - 2) // 2
    rpt = n_acc // NS
    nzc = rpt // B
    mesh = plsc.VectorSubcoreMesh(core_axis_name="c", subcore_axis_name="s")

    def body(tab, sr, dr, zd_h, pout, src_v, dst_v, rows0, rows1,
             acc_sh, gsem0, gsem1, ssem0, ssem1):
        c = lax.axis_index("c")
        s = lax.axis_index("s")
        wid = c * NS + s
        stripe = s * rpt

        pltpu.sync_copy(sr.at[wid], src_v)
        pltpu.sync_copy(dr.at[wid], dst_v)

        pltpu.sync_copy(zd_h, rows0)
        for i in range(nzc):
            pltpu.sync_copy(rows0, acc_sh.at[pl.ds(stripe + i * B, B)])
        plsc.subcore_barrier()

        def step(t, carry):
            pltpu.async_copy(tab.at[src_v.at[t]], rows0, gsem0).wait()
            pltpu.sync_copy(rows0, acc_sh.at[dst_v.at[t]], add=True)
            return carry

        lax.fori_loop(0, nl * 2, step, 0)
        plsc.subcore_barrier()

        for i in range(nzc):
            r0 = stripe + i * B
            buf = rows0 if i % 2 == 0 else rows1
            pltpu.sync_copy(acc_sh.at[pl.ds(r0, B)], buf)
            pltpu.sync_copy(buf, pout.at[pl.ds(c * n_acc + r0, B)])

    fn = pl.kernel(
        body,
        out_type=[jax.ShapeDtypeStruct((NC * n_acc, d), jnp.float32)],
        mesh=mesh,
        scratch_types=[
            pltpu.VMEM((nbs, B), jnp.int32),
            pltpu.VMEM((nbs, B), jnp.int32),
            pltpu.VMEM((B, d), jnp.float32),
            pltpu.VMEM((B, d), jnp.float32),
            pltpu.VMEM_SHARED((n_acc, d), jnp.float32),
            pltpu.SemaphoreType.DMA,
            pltpu.SemaphoreType.DMA,
            pltpu.SemaphoreType.DMA,
            pltpu.SemaphoreType.DMA,
        ],
        compiler_params=pltpu.CompilerParams(use_tc_tiling_on_sc=False),
    )
    return fn(table, srcr, dstr, zeros_d)[0]


def kernel(x, edge_index, W_l1, b_l1, W_r1, W_l2, b_l2, W_r2):
    n, d_in = x.shape
    d_hid = W_l1.shape[1]
    d_out = W_l2.shape[1]
    e = edge_index.shape[1]

    blk = 1024
    n_acc = ((n + (NS * B) - 1) // (NS * B)) * (NS * B)
    e_tile = ((e + (NW * B) - 1) // (NW * B)) * B
    nb = e_tile // B
    e_pad = NW * e_tile

    src = edge_index[0].astype(jnp.int32)
    dst = edge_index[1].astype(jnp.int32)
    spare = n_acc - n
    pad_dst = n + (jnp.arange(e_pad - e, dtype=jnp.int32) % spare)
    src = jnp.concatenate([src, jnp.zeros((e_pad - e,), jnp.int32)])
    dst = jnp.concatenate([dst, pad_dst])
    srcr = src.reshape(NW, nb, B)
    dstr = dst.reshape(NW, nb, B)
    pad_b = 3 if nb % 2 else 4
    dummy_dst = n + (jnp.arange(pad_b * B, dtype=jnp.int32) % spare)
    dummy_dst = jnp.broadcast_to(dummy_dst.reshape(1, pad_b, B),
                                 (NW, pad_b, B))
    srcr = jnp.concatenate(
        [srcr, jnp.zeros((NW, pad_b, B), jnp.int32)], axis=1)
    dstr = jnp.concatenate([dstr, dummy_dst], axis=1)

    zeros_e = jnp.zeros((B, d_hid + DEGW), jnp.float32)
    zeros_o = jnp.zeros((B, d_out), jnp.float32)

    wcat1 = jnp.concatenate([W_l1, W_r1], axis=1)
    y1e, r1 = _tc_l1(x, wcat1, n_acc, blk)

    pflat = _sc_aggregate(y1e, srcr, dstr, zeros_e, n_acc)

    wcat2 = jnp.concatenate([W_l2, W_r2], axis=1)
    y2, r2, rd = _tc_mid(pflat, r1, b_l1.reshape(1, d_hid), wcat2,
                         n_acc, blk)

    qflat = _sc_aggregate(y2, srcr, dstr, zeros_o, n_acc)

    return _tc_fin(qflat, rd, r2, b_l2.reshape(1, d_out), n, n_acc, blk)


# speedup vs baseline: 1.0001x; 1.0001x over previous
import jax
import jax.numpy as jnp
from jax import lax
from jax.experimental import pallas as pl
from jax.experimental.pallas import tpu as pltpu
from jax.experimental.pallas import tpu_sc as plsc

NC = 2
NS = 16
NW = NC * NS
B = 128
DEGW = 16


def _l1_body(x_ref, w_ref, y1e_ref, r1_ref):
    yr = jnp.dot(x_ref[...], w_ref[...], preferred_element_type=jnp.float32)
    dh = w_ref.shape[1] // 2
    y1e_ref[:, :dh] = yr[:, :dh]
    y1e_ref[:, dh:] = jnp.ones((yr.shape[0], DEGW), jnp.float32)
    r1_ref[...] = yr[:, dh:]


def _tc_l1(x, w, n_acc, blk):
    d = x.shape[1]
    dh = w.shape[1] // 2
    grid = (n_acc // blk,)
    return pl.pallas_call(
        _l1_body,
        grid=grid,
        in_specs=[pl.BlockSpec((blk, d), lambda i: (i, 0)),
                  pl.BlockSpec((d, 2 * dh), lambda i: (0, 0))],
        out_specs=[pl.BlockSpec((blk, dh + DEGW), lambda i: (i, 0)),
                   pl.BlockSpec((blk, dh), lambda i: (i, 0))],
        out_shape=[jax.ShapeDtypeStruct((n_acc, dh + DEGW), jnp.float32),
                   jax.ShapeDtypeStruct((n_acc, dh), jnp.float32)],
    )(x, w)


def _mid_body(p0_ref, p1_ref, r1_ref, b1_ref, w_ref, y2_ref, r2_ref, rd_ref):
    dh = r1_ref.shape[1]
    deg = p0_ref[:, dh:dh + 1] + p1_ref[:, dh:dh + 1]
    rdeg = 1.0 / jnp.maximum(deg, 1.0)
    h = (p0_ref[:, :dh] + p1_ref[:, :dh]) * rdeg + b1_ref[...] + r1_ref[...]
    h = jnp.maximum(h, 0.0)
    yr = jnp.dot(h, w_ref[...], preferred_element_type=jnp.float32)
    do = w_ref.shape[1] // 2
    y2_ref[...] = yr[:, :do]
    r2_ref[...] = yr[:, do:]
    rd_ref[...] = jnp.broadcast_to(rdeg, (rdeg.shape[0], DEGW))


def _tc_mid(pflat, r1, b1, w, n_acc, blk):
    dh = r1.shape[1]
    do2 = w.shape[1]
    do = do2 // 2
    nblk = n_acc // blk
    grid = (nblk,)
    return pl.pallas_call(
        _mid_body,
        grid=grid,
        in_specs=[pl.BlockSpec((blk, dh + DEGW), lambda i: (i, 0)),
                  pl.BlockSpec((blk, dh + DEGW), lambda i: (i + nblk, 0)),
                  pl.BlockSpec((blk, dh), lambda i: (i, 0)),
                  pl.BlockSpec((1, dh), lambda i: (0, 0)),
                  pl.BlockSpec((dh, do2), lambda i: (0, 0))],
        out_specs=[pl.BlockSpec((blk, do), lambda i: (i, 0)),
                   pl.BlockSpec((blk, do), lambda i: (i, 0)),
                   pl.BlockSpec((blk, DEGW), lambda i: (i, 0))],
        out_shape=[jax.ShapeDtypeStruct((n_acc, do), jnp.float32),
                   jax.ShapeDtypeStruct((n_acc, do), jnp.float32),
                   jax.ShapeDtypeStruct((n_acc, DEGW), jnp.float32)],
    )(pflat, pflat, r1, b1, w)


def _fin_body(q0_ref, q1_ref, rd_ref, r2_ref, b2_ref, o_ref):
    rdeg = rd_ref[:, :1]
    o_ref[...] = (q0_ref[...] + q1_ref[...]) * rdeg + b2_ref[...] + r2_ref[...]


def _tc_fin(qflat, rd, r2, b2, n, n_acc, blk):
    do = r2.shape[1]
    nblk = n_acc // blk
    grid = (nblk,)
    return pl.pallas_call(
        _fin_body,
        grid=grid,
        in_specs=[pl.BlockSpec((blk, do), lambda i: (i, 0)),
                  pl.BlockSpec((blk, do), lambda i: (i + nblk, 0)),
                  pl.BlockSpec((blk, DEGW), lambda i: (i, 0)),
                  pl.BlockSpec((blk, do), lambda i: (i, 0)),
                  pl.BlockSpec((1, do), lambda i: (0, 0))],
        out_specs=pl.BlockSpec((blk, do), lambda i: (i, 0)),
        out_shape=jax.ShapeDtypeStruct((n, do), jnp.float32),
    )(qflat, qflat, rd, r2, b2)


def _sc_aggregate(table, srcr, dstr, zeros_d, n_acc):
    d = table.shape[1]
    nbs = srcr.shape[1]
    nl = (nbs - 2) // 2
    rpt = n_acc // NS
    nzc = rpt // B
    mesh = plsc.VectorSubcoreMesh(core_axis_name="c", subcore_axis_name="s")

    def body(tab, sr, dr, zd_h, pout, src_v, dst_v, rows0,
             acc_sh, gsem0):
        c = lax.axis_index("c")
        s = lax.axis_index("s")
        wid = c * NS + s
        stripe = s * rpt

        pltpu.sync_copy(sr.at[wid], src_v)
        pltpu.sync_copy(dr.at[wid], dst_v)

        pltpu.sync_copy(zd_h, rows0)
        for i in range(nzc):
            pltpu.sync_copy(rows0, acc_sh.at[pl.ds(stripe + i * B, B)])
        plsc.subcore_barrier()

        def step(t, carry):
            pltpu.async_copy(tab.at[src_v.at[t]], rows0, gsem0).wait()
            pltpu.sync_copy(rows0, acc_sh.at[dst_v.at[t]], add=True)
            return carry

        lax.fori_loop(0, nl * 2, step, 0)
        plsc.subcore_barrier()

        for i in range(nzc):
            r0 = stripe + i * B
            pltpu.sync_copy(acc_sh.at[pl.ds(r0, B)], rows0)
            pltpu.sync_copy(rows0, pout.at[pl.ds(c * n_acc + r0, B)])

    fn = pl.kernel(
        body,
        out_type=[jax.ShapeDtypeStruct((NC * n_acc, d), jnp.float32)],
        mesh=mesh,
        scratch_types=[
            pltpu.VMEM((nbs, B), jnp.int32),
            pltpu.VMEM((nbs, B), jnp.int32),
            pltpu.VMEM((B, d), jnp.float32),
            pltpu.VMEM_SHARED((n_acc, d), jnp.float32),
            pltpu.SemaphoreType.DMA,
        ],
        compiler_params=pltpu.CompilerParams(use_tc_tiling_on_sc=False),
    )
    return fn(table, srcr, dstr, zeros_d)[0]


def kernel(x, edge_index, W_l1, b_l1, W_r1, W_l2, b_l2, W_r2):
    n, d_in = x.shape
    d_hid = W_l1.shape[1]
    d_out = W_l2.shape[1]
    e = edge_index.shape[1]

    blk = 1024
    n_acc = ((n + (NS * B) - 1) // (NS * B)) * (NS * B)
    e_tile = ((e + (NW * B) - 1) // (NW * B)) * B
    nb = e_tile // B
    e_pad = NW * e_tile

    src = edge_index[0].astype(jnp.int32)
    dst = edge_index[1].astype(jnp.int32)
    spare = n_acc - n
    pad_dst = n + (jnp.arange(e_pad - e, dtype=jnp.int32) % spare)
    src = jnp.concatenate([src, jnp.zeros((e_pad - e,), jnp.int32)])
    dst = jnp.concatenate([dst, pad_dst])
    srcr = src.reshape(NW, nb, B)
    dstr = dst.reshape(NW, nb, B)
    pad_b = 3 if nb % 2 else 4
    dummy_dst = n + (jnp.arange(pad_b * B, dtype=jnp.int32) % spare)
    dummy_dst = jnp.broadcast_to(dummy_dst.reshape(1, pad_b, B),
                                 (NW, pad_b, B))
    srcr = jnp.concatenate(
        [srcr, jnp.zeros((NW, pad_b, B), jnp.int32)], axis=1)
    dstr = jnp.concatenate([dstr, dummy_dst], axis=1)

    zeros_e = jnp.zeros((B, d_hid + DEGW), jnp.float32)
    zeros_o = jnp.zeros((B, d_out), jnp.float32)

    wcat1 = jnp.concatenate([W_l1, W_r1], axis=1)
    y1e, r1 = _tc_l1(x, wcat1, n_acc, blk)

    pflat = _sc_aggregate(y1e, srcr, dstr, zeros_e, n_acc)

    wcat2 = jnp.concatenate([W_l2, W_r2], axis=1)
    y2, r2, rd = _tc_mid(pflat, r1, b_l1.reshape(1, d_hid), wcat2,
                         n_acc, blk)

    qflat = _sc_aggregate(y2, srcr, dstr, zeros_o, n_acc)

    return _tc_fin(qflat, rd, r2, b_l2.reshape(1, d_out), n, n_acc, blk)


# speedup vs baseline: 1.0683x; 1.0682x over previous
import jax
import jax.numpy as jnp
from jax import lax
from jax.experimental import pallas as pl
from jax.experimental.pallas import tpu as pltpu
from jax.experimental.pallas import tpu_sc as plsc

NC = 2
NS = 16
NW = NC * NS
B = 128
DEGW = 16


def _l1_body(x_ref, w_ref, y1e_ref, r1_ref):
    yr = jnp.dot(x_ref[...], w_ref[...], preferred_element_type=jnp.float32)
    dh = w_ref.shape[1] // 2
    y1e_ref[:, :dh] = yr[:, :dh]
    y1e_ref[:, dh:] = jnp.ones((yr.shape[0], DEGW), jnp.float32)
    r1_ref[...] = yr[:, dh:]


def _tc_l1(x, w, n_acc, blk):
    d = x.shape[1]
    dh = w.shape[1] // 2
    grid = (n_acc // blk,)
    return pl.pallas_call(
        _l1_body,
        grid=grid,
        in_specs=[pl.BlockSpec((blk, d), lambda i: (i, 0)),
                  pl.BlockSpec((d, 2 * dh), lambda i: (0, 0))],
        out_specs=[pl.BlockSpec((blk, dh + DEGW), lambda i: (i, 0)),
                   pl.BlockSpec((blk, dh), lambda i: (i, 0))],
        out_shape=[jax.ShapeDtypeStruct((n_acc, dh + DEGW), jnp.float32),
                   jax.ShapeDtypeStruct((n_acc, dh), jnp.float32)],
    )(x, w)


def _mid_body(p0_ref, p1_ref, r1_ref, b1_ref, w_ref, y2_ref, r2_ref, rd_ref):
    dh = r1_ref.shape[1]
    deg = p0_ref[:, dh:dh + 1] + p1_ref[:, dh:dh + 1]
    rdeg = 1.0 / jnp.maximum(deg, 1.0)
    h = (p0_ref[:, :dh] + p1_ref[:, :dh]) * rdeg + b1_ref[...] + r1_ref[...]
    h = jnp.maximum(h, 0.0)
    yr = jnp.dot(h, w_ref[...], preferred_element_type=jnp.float32)
    do = w_ref.shape[1] // 2
    y2_ref[...] = yr[:, :do]
    r2_ref[...] = yr[:, do:]
    rd_ref[...] = jnp.broadcast_to(rdeg, (rdeg.shape[0], DEGW))


def _tc_mid(pflat, r1, b1, w, n_acc, blk):
    dh = r1.shape[1]
    do2 = w.shape[1]
    do = do2 // 2
    nblk = n_acc // blk
    grid = (nblk,)
    return pl.pallas_call(
        _mid_body,
        grid=grid,
        in_specs=[pl.BlockSpec((blk, dh + DEGW), lambda i: (i, 0)),
                  pl.BlockSpec((blk, dh + DEGW), lambda i: (i + nblk, 0)),
                  pl.BlockSpec((blk, dh), lambda i: (i, 0)),
                  pl.BlockSpec((1, dh), lambda i: (0, 0)),
                  pl.BlockSpec((dh, do2), lambda i: (0, 0))],
        out_specs=[pl.BlockSpec((blk, do), lambda i: (i, 0)),
                   pl.BlockSpec((blk, do), lambda i: (i, 0)),
                   pl.BlockSpec((blk, DEGW), lambda i: (i, 0))],
        out_shape=[jax.ShapeDtypeStruct((n_acc, do), jnp.float32),
                   jax.ShapeDtypeStruct((n_acc, do), jnp.float32),
                   jax.ShapeDtypeStruct((n_acc, DEGW), jnp.float32)],
    )(pflat, pflat, r1, b1, w)


def _fin_body(q0_ref, q1_ref, rd_ref, r2_ref, b2_ref, o_ref):
    rdeg = rd_ref[:, :1]
    o_ref[...] = (q0_ref[...] + q1_ref[...]) * rdeg + b2_ref[...] + r2_ref[...]


def _tc_fin(qflat, rd, r2, b2, n, n_acc, blk):
    do = r2.shape[1]
    nblk = n_acc // blk
    grid = (nblk,)
    return pl.pallas_call(
        _fin_body,
        grid=grid,
        in_specs=[pl.BlockSpec((blk, do), lambda i: (i, 0)),
                  pl.BlockSpec((blk, do), lambda i: (i + nblk, 0)),
                  pl.BlockSpec((blk, DEGW), lambda i: (i, 0)),
                  pl.BlockSpec((blk, do), lambda i: (i, 0)),
                  pl.BlockSpec((1, do), lambda i: (0, 0))],
        out_specs=pl.BlockSpec((blk, do), lambda i: (i, 0)),
        out_shape=jax.ShapeDtypeStruct((n, do), jnp.float32),
    )(qflat, qflat, rd, r2, b2)


def _sc_aggregate(table, srcr, dstr, zeros_d, n_acc):
    d = table.shape[1]
    nbs = srcr.shape[1]
    nl = (nbs - 2) // 2
    rpt = n_acc // NS
    nzc = rpt // B
    mesh = plsc.VectorSubcoreMesh(core_axis_name="c", subcore_axis_name="s")

    def body(tab, sr, dr, zd_h, pout, src_v, dst_v, rows0, rows1, rows2,
             acc_sh, gsem, ssem):
        c = lax.axis_index("c")
        s = lax.axis_index("s")
        wid = c * NS + s
        stripe = s * rpt

        pltpu.sync_copy(sr.at[wid], src_v)
        pltpu.sync_copy(dr.at[wid], dst_v)

        pltpu.sync_copy(zd_h, rows0)
        for i in range(nzc):
            pltpu.sync_copy(rows0, acc_sh.at[pl.ds(stripe + i * B, B)])
        plsc.subcore_barrier()

        bufs = (rows0, rows1, rows2)
        nbat = nl * 2
        gd = pltpu.async_copy(tab.at[src_v.at[0]], bufs[0], gsem)
        sd = None
        for t in range(nbat):
            gd.wait()
            if t + 1 < nbat:
                gd = pltpu.async_copy(tab.at[src_v.at[t + 1]],
                                      bufs[(t + 1) % 3], gsem)
            if sd is not None:
                sd.wait()
            sd = pltpu.async_copy(bufs[t % 3], acc_sh.at[dst_v.at[t]],
                                  ssem, add=True)
        sd.wait()
        plsc.subcore_barrier()

        for i in range(nzc):
            r0 = stripe + i * B
            pltpu.sync_copy(acc_sh.at[pl.ds(r0, B)], rows0)
            pltpu.sync_copy(rows0, pout.at[pl.ds(c * n_acc + r0, B)])

    fn = pl.kernel(
        body,
        out_type=[jax.ShapeDtypeStruct((NC * n_acc, d), jnp.float32)],
        mesh=mesh,
        scratch_types=[
            pltpu.VMEM((nbs, B), jnp.int32),
            pltpu.VMEM((nbs, B), jnp.int32),
            pltpu.VMEM((B, d), jnp.float32),
            pltpu.VMEM((B, d), jnp.float32),
            pltpu.VMEM((B, d), jnp.float32),
            pltpu.VMEM_SHARED((n_acc, d), jnp.float32),
            pltpu.SemaphoreType.DMA,
            pltpu.SemaphoreType.DMA,
        ],
        compiler_params=pltpu.CompilerParams(use_tc_tiling_on_sc=False),
    )
    return fn(table, srcr, dstr, zeros_d)[0]


def kernel(x, edge_index, W_l1, b_l1, W_r1, W_l2, b_l2, W_r2):
    n, d_in = x.shape
    d_hid = W_l1.shape[1]
    d_out = W_l2.shape[1]
    e = edge_index.shape[1]

    blk = 1024
    n_acc = ((n + (NS * B) - 1) // (NS * B)) * (NS * B)
    e_tile = ((e + (NW * B) - 1) // (NW * B)) * B
    nb = e_tile // B
    e_pad = NW * e_tile

    src = edge_index[0].astype(jnp.int32)
    dst = edge_index[1].astype(jnp.int32)
    spare = n_acc - n
    pad_dst = n + (jnp.arange(e_pad - e, dtype=jnp.int32) % spare)
    src = jnp.concatenate([src, jnp.zeros((e_pad - e,), jnp.int32)])
    dst = jnp.concatenate([dst, pad_dst])
    srcr = src.reshape(NW, nb, B)
    dstr = dst.reshape(NW, nb, B)
    pad_b = 3 if nb % 2 else 4
    dummy_dst = n + (jnp.arange(pad_b * B, dtype=jnp.int32) % spare)
    dummy_dst = jnp.broadcast_to(dummy_dst.reshape(1, pad_b, B),
                                 (NW, pad_b, B))
    srcr = jnp.concatenate(
        [srcr, jnp.zeros((NW, pad_b, B), jnp.int32)], axis=1)
    dstr = jnp.concatenate([dstr, dummy_dst], axis=1)

    zeros_e = jnp.zeros((B, d_hid + DEGW), jnp.float32)
    zeros_o = jnp.zeros((B, d_out), jnp.float32)

    wcat1 = jnp.concatenate([W_l1, W_r1], axis=1)
    y1e, r1 = _tc_l1(x, wcat1, n_acc, blk)

    pflat = _sc_aggregate(y1e, srcr, dstr, zeros_e, n_acc)

    wcat2 = jnp.concatenate([W_l2, W_r2], axis=1)
    y2, r2, rd = _tc_mid(pflat, r1, b_l1.reshape(1, d_hid), wcat2,
                         n_acc, blk)

    qflat = _sc_aggregate(y2, srcr, dstr, zeros_o, n_acc)

    return _tc_fin(qflat, rd, r2, b_l2.reshape(1, d_out), n, n_acc, blk)


# speedup vs baseline: 1.5990x; 1.4967x over previous
import jax
import jax.numpy as jnp
from jax import lax
from jax.experimental import pallas as pl
from jax.experimental.pallas import tpu as pltpu
from jax.experimental.pallas import tpu_sc as plsc

NC = 2
NS = 16
NW = NC * NS
B = 128
DEGW = 16


def _l1_body(x_ref, w_ref, y1e_ref, r1_ref):
    yr = jnp.dot(x_ref[...], w_ref[...], preferred_element_type=jnp.float32)
    dh = w_ref.shape[1] // 2
    y1e_ref[:, :dh] = yr[:, :dh]
    y1e_ref[:, dh:] = jnp.ones((yr.shape[0], DEGW), jnp.float32)
    r1_ref[...] = yr[:, dh:]


def _tc_l1(x, w, n_acc, blk):
    d = x.shape[1]
    dh = w.shape[1] // 2
    grid = (n_acc // blk,)
    return pl.pallas_call(
        _l1_body,
        grid=grid,
        in_specs=[pl.BlockSpec((blk, d), lambda i: (i, 0)),
                  pl.BlockSpec((d, 2 * dh), lambda i: (0, 0))],
        out_specs=[pl.BlockSpec((blk, dh + DEGW), lambda i: (i, 0)),
                   pl.BlockSpec((blk, dh), lambda i: (i, 0))],
        out_shape=[jax.ShapeDtypeStruct((n_acc, dh + DEGW), jnp.float32),
                   jax.ShapeDtypeStruct((n_acc, dh), jnp.float32)],
    )(x, w)


def _mid_body(p0_ref, p1_ref, r1_ref, b1_ref, w_ref, y2_ref, r2_ref, rd_ref):
    dh = r1_ref.shape[1]
    deg = p0_ref[:, dh:dh + 1] + p1_ref[:, dh:dh + 1]
    rdeg = 1.0 / jnp.maximum(deg, 1.0)
    h = (p0_ref[:, :dh] + p1_ref[:, :dh]) * rdeg + b1_ref[...] + r1_ref[...]
    h = jnp.maximum(h, 0.0)
    yr = jnp.dot(h, w_ref[...], preferred_element_type=jnp.float32)
    do = w_ref.shape[1] // 2
    y2_ref[...] = yr[:, :do]
    r2_ref[...] = yr[:, do:]
    rd_ref[...] = jnp.broadcast_to(rdeg, (rdeg.shape[0], DEGW))


def _tc_mid(pflat, r1, b1, w, n_acc, blk):
    dh = r1.shape[1]
    do2 = w.shape[1]
    do = do2 // 2
    nblk = n_acc // blk
    grid = (nblk,)
    return pl.pallas_call(
        _mid_body,
        grid=grid,
        in_specs=[pl.BlockSpec((blk, dh + DEGW), lambda i: (i, 0)),
                  pl.BlockSpec((blk, dh + DEGW), lambda i: (i + nblk, 0)),
                  pl.BlockSpec((blk, dh), lambda i: (i, 0)),
                  pl.BlockSpec((1, dh), lambda i: (0, 0)),
                  pl.BlockSpec((dh, do2), lambda i: (0, 0))],
        out_specs=[pl.BlockSpec((blk, do), lambda i: (i, 0)),
                   pl.BlockSpec((blk, do), lambda i: (i, 0)),
                   pl.BlockSpec((blk, DEGW), lambda i: (i, 0))],
        out_shape=[jax.ShapeDtypeStruct((n_acc, do), jnp.float32),
                   jax.ShapeDtypeStruct((n_acc, do), jnp.float32),
                   jax.ShapeDtypeStruct((n_acc, DEGW), jnp.float32)],
    )(pflat, pflat, r1, b1, w)


def _fin_body(q0_ref, q1_ref, rd_ref, r2_ref, b2_ref, o_ref):
    rdeg = rd_ref[:, :1]
    o_ref[...] = (q0_ref[...] + q1_ref[...]) * rdeg + b2_ref[...] + r2_ref[...]


def _tc_fin(qflat, rd, r2, b2, n, n_acc, blk):
    do = r2.shape[1]
    nblk = n_acc // blk
    grid = (nblk,)
    return pl.pallas_call(
        _fin_body,
        grid=grid,
        in_specs=[pl.BlockSpec((blk, do), lambda i: (i, 0)),
                  pl.BlockSpec((blk, do), lambda i: (i + nblk, 0)),
                  pl.BlockSpec((blk, DEGW), lambda i: (i, 0)),
                  pl.BlockSpec((blk, do), lambda i: (i, 0)),
                  pl.BlockSpec((1, do), lambda i: (0, 0))],
        out_specs=pl.BlockSpec((blk, do), lambda i: (i, 0)),
        out_shape=jax.ShapeDtypeStruct((n, do), jnp.float32),
    )(qflat, qflat, rd, r2, b2)


def _sc_aggregate(table, srcr, dstr, zeros_d, n_acc):
    d = table.shape[1]
    nbs = srcr.shape[1]
    rpt = n_acc // NS
    nzc = rpt // B
    mesh = plsc.VectorSubcoreMesh(core_axis_name="c", subcore_axis_name="s")

    def body(tab, sr, dr, zd_h, pout, src_v, dst_v, rows0, rows1, rows2,
             acc_sh, gsem, ssem):
        c = lax.axis_index("c")
        s = lax.axis_index("s")
        wid = c * NS + s
        stripe = s * rpt

        pltpu.sync_copy(sr.at[wid], src_v)
        pltpu.sync_copy(dr.at[wid], dst_v)

        pltpu.sync_copy(zd_h, rows0)
        for i in range(nzc):
            pltpu.sync_copy(rows0, acc_sh.at[pl.ds(stripe + i * B, B)])
        plsc.subcore_barrier()

        bufs = (rows0, rows1, rows2)
        nbat = nbs
        gd = pltpu.async_copy(tab.at[src_v.at[0]], bufs[0], gsem)
        sd = None
        for t in range(nbat):
            gd.wait()
            if t + 1 < nbat:
                gd = pltpu.async_copy(tab.at[src_v.at[t + 1]],
                                      bufs[(t + 1) % 3], gsem)
            if sd is not None:
                sd.wait()
            sd = pltpu.async_copy(bufs[t % 3], acc_sh.at[dst_v.at[t]],
                                  ssem, add=True)
        sd.wait()
        plsc.subcore_barrier()

        for i in range(nzc):
            r0 = stripe + i * B
            pltpu.sync_copy(acc_sh.at[pl.ds(r0, B)], rows0)
            pltpu.sync_copy(rows0, pout.at[pl.ds(c * n_acc + r0, B)])

    fn = pl.kernel(
        body,
        out_type=[jax.ShapeDtypeStruct((NC * n_acc, d), jnp.float32)],
        mesh=mesh,
        scratch_types=[
            pltpu.VMEM((nbs, B), jnp.int32),
            pltpu.VMEM((nbs, B), jnp.int32),
            pltpu.VMEM((B, d), jnp.float32),
            pltpu.VMEM((B, d), jnp.float32),
            pltpu.VMEM((B, d), jnp.float32),
            pltpu.VMEM_SHARED((n_acc, d), jnp.float32),
            pltpu.SemaphoreType.DMA,
            pltpu.SemaphoreType.DMA,
        ],
        compiler_params=pltpu.CompilerParams(use_tc_tiling_on_sc=False),
    )
    return fn(table, srcr, dstr, zeros_d)[0]


def kernel(x, edge_index, W_l1, b_l1, W_r1, W_l2, b_l2, W_r2):
    n, d_in = x.shape
    d_hid = W_l1.shape[1]
    d_out = W_l2.shape[1]
    e = edge_index.shape[1]

    blk = 1024
    n_acc = ((n + (NS * B) - 1) // (NS * B)) * (NS * B)
    e_tile = ((e + (NW * B) - 1) // (NW * B)) * B
    nb = e_tile // B
    e_pad = NW * e_tile

    src = edge_index[0].astype(jnp.int32)
    dst = edge_index[1].astype(jnp.int32)
    spare = n_acc - n
    pad_dst = n + (jnp.arange(e_pad - e, dtype=jnp.int32) % spare)
    src = jnp.concatenate([src, jnp.zeros((e_pad - e,), jnp.int32)])
    dst = jnp.concatenate([dst, pad_dst])
    srcr = src.reshape(NW, nb, B)
    dstr = dst.reshape(NW, nb, B)

    zeros_e = jnp.zeros((B, d_hid + DEGW), jnp.float32)
    zeros_o = jnp.zeros((B, d_out), jnp.float32)

    wcat1 = jnp.concatenate([W_l1, W_r1], axis=1)
    y1e, r1 = _tc_l1(x, wcat1, n_acc, blk)

    pflat = _sc_aggregate(y1e, srcr, dstr, zeros_e, n_acc)

    wcat2 = jnp.concatenate([W_l2, W_r2], axis=1)
    y2, r2, rd = _tc_mid(pflat, r1, b_l1.reshape(1, d_hid), wcat2,
                         n_acc, blk)

    qflat = _sc_aggregate(y2, srcr, dstr, zeros_o, n_acc)

    return _tc_fin(qflat, rd, r2, b_l2.reshape(1, d_out), n, n_acc, blk)


# speedup vs baseline: 1.8672x; 1.1677x over previous
import jax
import jax.numpy as jnp
from jax import lax
from jax.experimental import pallas as pl
from jax.experimental.pallas import tpu as pltpu
from jax.experimental.pallas import tpu_sc as plsc

NC = 2
NS = 16
NW = NC * NS
B = 128
DEGW = 16


def _l1_body(x_ref, w_ref, y1e_ref, r1_ref):
    yr = jnp.dot(x_ref[...], w_ref[...], preferred_element_type=jnp.float32)
    dh = w_ref.shape[1] // 2
    y1e_ref[:, :dh] = yr[:, :dh]
    y1e_ref[:, dh:] = jnp.ones((yr.shape[0], DEGW), jnp.float32)
    r1_ref[...] = yr[:, dh:]


def _tc_l1(x, w, n_acc, blk):
    d = x.shape[1]
    dh = w.shape[1] // 2
    grid = (n_acc // blk,)
    return pl.pallas_call(
        _l1_body,
        grid=grid,
        in_specs=[pl.BlockSpec((blk, d), lambda i: (i, 0)),
                  pl.BlockSpec((d, 2 * dh), lambda i: (0, 0))],
        out_specs=[pl.BlockSpec((blk, dh + DEGW), lambda i: (i, 0)),
                   pl.BlockSpec((blk, dh), lambda i: (i, 0))],
        out_shape=[jax.ShapeDtypeStruct((n_acc, dh + DEGW), jnp.float32),
                   jax.ShapeDtypeStruct((n_acc, dh), jnp.float32)],
    )(x, w)


def _mid_body(p0_ref, p1_ref, r1_ref, b1_ref, w_ref, y2_ref, r2_ref, rd_ref):
    dh = r1_ref.shape[1]
    deg = p0_ref[:, dh:dh + 1] + p1_ref[:, dh:dh + 1]
    rdeg = 1.0 / jnp.maximum(deg, 1.0)
    h = (p0_ref[:, :dh] + p1_ref[:, :dh]) * rdeg + b1_ref[...] + r1_ref[...]
    h = jnp.maximum(h, 0.0)
    yr = jnp.dot(h, w_ref[...], preferred_element_type=jnp.float32)
    do = w_ref.shape[1] // 2
    y2_ref[...] = yr[:, :do]
    r2_ref[...] = yr[:, do:]
    rd_ref[...] = jnp.broadcast_to(rdeg, (rdeg.shape[0], DEGW))


def _tc_mid(pflat, r1, b1, w, n_acc, blk):
    dh = r1.shape[1]
    do2 = w.shape[1]
    do = do2 // 2
    nblk = n_acc // blk
    grid = (nblk,)
    return pl.pallas_call(
        _mid_body,
        grid=grid,
        in_specs=[pl.BlockSpec((blk, dh + DEGW), lambda i: (i, 0)),
                  pl.BlockSpec((blk, dh + DEGW), lambda i: (i + nblk, 0)),
                  pl.BlockSpec((blk, dh), lambda i: (i, 0)),
                  pl.BlockSpec((1, dh), lambda i: (0, 0)),
                  pl.BlockSpec((dh, do2), lambda i: (0, 0))],
        out_specs=[pl.BlockSpec((blk, do), lambda i: (i, 0)),
                   pl.BlockSpec((blk, do), lambda i: (i, 0)),
                   pl.BlockSpec((blk, DEGW), lambda i: (i, 0))],
        out_shape=[jax.ShapeDtypeStruct((n_acc, do), jnp.float32),
                   jax.ShapeDtypeStruct((n_acc, do), jnp.float32),
                   jax.ShapeDtypeStruct((n_acc, DEGW), jnp.float32)],
    )(pflat, pflat, r1, b1, w)


def _fin_body(q0_ref, q1_ref, rd_ref, r2_ref, b2_ref, o_ref):
    rdeg = rd_ref[:, :1]
    o_ref[...] = (q0_ref[...] + q1_ref[...]) * rdeg + b2_ref[...] + r2_ref[...]


def _tc_fin(qflat, rd, r2, b2, n, n_acc, blk):
    do = r2.shape[1]
    nblk = n_acc // blk
    grid = (nblk,)
    return pl.pallas_call(
        _fin_body,
        grid=grid,
        in_specs=[pl.BlockSpec((blk, do), lambda i: (i, 0)),
                  pl.BlockSpec((blk, do), lambda i: (i + nblk, 0)),
                  pl.BlockSpec((blk, DEGW), lambda i: (i, 0)),
                  pl.BlockSpec((blk, do), lambda i: (i, 0)),
                  pl.BlockSpec((1, do), lambda i: (0, 0))],
        out_specs=pl.BlockSpec((blk, do), lambda i: (i, 0)),
        out_shape=jax.ShapeDtypeStruct((n, do), jnp.float32),
    )(qflat, qflat, rd, r2, b2)


def _sc_aggregate(table, srcr, dstr, zeros_d, n_acc, nb0, nb1):
    d = table.shape[1]
    rpt = n_acc // NS
    nzc = rpt // B
    mesh = plsc.VectorSubcoreMesh(core_axis_name="c", subcore_axis_name="s")

    def body(tab, sr, dr, zd_h, pout, src_v, dst_v, rows0, rows1, rows2,
             acc_sh, gsem, ssem):
        c = lax.axis_index("c")
        s = lax.axis_index("s")
        wid = c * NS + s
        stripe = s * rpt

        pltpu.sync_copy(sr.at[wid], src_v)
        pltpu.sync_copy(dr.at[wid], dst_v)

        pltpu.sync_copy(zd_h, rows0)
        for i in range(nzc):
            pltpu.sync_copy(rows0, acc_sh.at[pl.ds(stripe + i * B, B)])
        plsc.subcore_barrier()

        bufs = (rows0, rows1, rows2)

        def run_pipe(nbat):
            gd = pltpu.async_copy(tab.at[src_v.at[0]], bufs[0], gsem)
            sd = None
            for t in range(nbat):
                gd.wait()
                if t + 1 < nbat:
                    gd = pltpu.async_copy(tab.at[src_v.at[t + 1]],
                                          bufs[(t + 1) % 3], gsem)
                if sd is not None:
                    sd.wait()
                sd = pltpu.async_copy(bufs[t % 3], acc_sh.at[dst_v.at[t]],
                                      ssem, add=True)
            sd.wait()

        if nb0 == nb1:
            run_pipe(nb0)
        else:
            @pl.when(c == 0)
            def _():
                run_pipe(nb0)

            @pl.when(c == 1)
            def _():
                run_pipe(nb1)

        plsc.subcore_barrier()

        ods = {}
        for i in range(nzc):
            buf = bufs[i % 2]
            if i - 2 in ods:
                ods[i - 2].wait()
            pltpu.sync_copy(acc_sh.at[pl.ds(stripe + i * B, B)], buf)
            ods[i] = pltpu.async_copy(
                buf, pout.at[pl.ds(c * n_acc + stripe + i * B, B)], gsem)
        ods[nzc - 2].wait()
        ods[nzc - 1].wait()

    fn = pl.kernel(
        body,
        out_type=[jax.ShapeDtypeStruct((NC * n_acc, d), jnp.float32)],
        mesh=mesh,
        scratch_types=[
            pltpu.VMEM((nb0, B), jnp.int32),
            pltpu.VMEM((nb0, B), jnp.int32),
            pltpu.VMEM((B, d), jnp.float32),
            pltpu.VMEM((B, d), jnp.float32),
            pltpu.VMEM((B, d), jnp.float32),
            pltpu.VMEM_SHARED((n_acc, d), jnp.float32),
            pltpu.SemaphoreType.DMA,
            pltpu.SemaphoreType.DMA,
        ],
        compiler_params=pltpu.CompilerParams(use_tc_tiling_on_sc=False),
    )
    return fn(table, srcr, dstr, zeros_d)[0]


def _split_pack(vec, fill, nb0, nb1):
    e0 = NS * nb0 * B
    cap1 = NS * nb1 * B
    part0 = vec[:e0].reshape(NS, nb0, B)
    tail = vec[e0:]
    tail = jnp.concatenate([tail, fill[:cap1 - tail.shape[0]]])
    part1 = tail.reshape(NS, nb1, B)
    if nb1 < nb0:
        part1 = jnp.concatenate(
            [part1, jnp.zeros((NS, nb0 - nb1, B), jnp.int32)], axis=1)
    return jnp.concatenate([part0, part1], axis=0)


FRAC_L1 = 0.65
FRAC_L2 = 0.56


def kernel(x, edge_index, W_l1, b_l1, W_r1, W_l2, b_l2, W_r2):
    n, d_in = x.shape
    d_hid = W_l1.shape[1]
    d_out = W_l2.shape[1]
    e = edge_index.shape[1]

    blk = 1024
    n_acc = ((n + (NS * B) - 1) // (NS * B)) * (NS * B)
    tb = -(-e // (NS * B))
    nb0_1 = max(1, min(tb - 1, int(round(tb * FRAC_L1))))
    nb1_1 = tb - nb0_1
    nb0_2 = max(1, min(tb - 1, int(round(tb * FRAC_L2))))
    nb1_2 = tb - nb0_2

    src = edge_index[0].astype(jnp.int32)
    dst = edge_index[1].astype(jnp.int32)
    spare = n_acc - n
    src_fill = jnp.zeros((NS * B,), jnp.int32)
    dst_fill = n + (jnp.arange(NS * B, dtype=jnp.int32) % spare)
    srcr1 = _split_pack(src, src_fill, nb0_1, nb1_1)
    dstr1 = _split_pack(dst, dst_fill, nb0_1, nb1_1)
    srcr2 = _split_pack(src, src_fill, nb0_2, nb1_2)
    dstr2 = _split_pack(dst, dst_fill, nb0_2, nb1_2)

    zeros_e = jnp.zeros((B, d_hid + DEGW), jnp.float32)
    zeros_o = jnp.zeros((B, d_out), jnp.float32)

    wcat1 = jnp.concatenate([W_l1, W_r1], axis=1)
    y1e, r1 = _tc_l1(x, wcat1, n_acc, blk)

    pflat = _sc_aggregate(y1e, srcr1, dstr1, zeros_e, n_acc, nb0_1, nb1_1)

    wcat2 = jnp.concatenate([W_l2, W_r2], axis=1)
    y2, r2, rd = _tc_mid(pflat, r1, b_l1.reshape(1, d_hid), wcat2,
                         n_acc, blk)

    qflat = _sc_aggregate(y2, srcr2, dstr2, zeros_o, n_acc, nb0_2, nb1_2)

    return _tc_fin(qflat, rd, r2, b_l2.reshape(1, d_out), n, n_acc, blk)


# speedup vs baseline: 2.1318x; 1.1417x over previous
import jax
import jax.numpy as jnp
from jax import lax
from jax.experimental import pallas as pl
from jax.experimental.pallas import tpu as pltpu
from jax.experimental.pallas import tpu_sc as plsc

NC = 2
NS = 16
NW = NC * NS
B = 128
DEGW = 16


def _l1_body(x_ref, w_ref, y1e_ref, r1_ref):
    yr = jnp.dot(x_ref[...], w_ref[...], preferred_element_type=jnp.float32)
    dh = w_ref.shape[1] // 2
    y1e_ref[:, :dh] = yr[:, :dh]
    y1e_ref[:, dh:] = jnp.ones((yr.shape[0], DEGW), jnp.float32)
    r1_ref[...] = yr[:, dh:]


def _tc_l1(x, w, n_acc, blk):
    d = x.shape[1]
    dh = w.shape[1] // 2
    grid = (n_acc // blk,)
    return pl.pallas_call(
        _l1_body,
        grid=grid,
        in_specs=[pl.BlockSpec((blk, d), lambda i: (i, 0)),
                  pl.BlockSpec((d, 2 * dh), lambda i: (0, 0))],
        out_specs=[pl.BlockSpec((blk, dh + DEGW), lambda i: (i, 0)),
                   pl.BlockSpec((blk, dh), lambda i: (i, 0))],
        out_shape=[jax.ShapeDtypeStruct((n_acc, dh + DEGW), jnp.float32),
                   jax.ShapeDtypeStruct((n_acc, dh), jnp.float32)],
    )(x, w)


def _mid_body(p0_ref, p1_ref, r1_ref, b1_ref, w_ref, y2_ref, r2_ref, rd_ref):
    dh = r1_ref.shape[1]
    deg = p0_ref[:, dh:dh + 1] + p1_ref[:, dh:dh + 1]
    rdeg = 1.0 / jnp.maximum(deg, 1.0)
    h = (p0_ref[:, :dh] + p1_ref[:, :dh]) * rdeg + b1_ref[...] + r1_ref[...]
    h = jnp.maximum(h, 0.0)
    yr = jnp.dot(h, w_ref[...], preferred_element_type=jnp.float32)
    do = w_ref.shape[1] // 2
    y2_ref[...] = yr[:, :do]
    r2_ref[...] = yr[:, do:]
    rd_ref[...] = jnp.broadcast_to(rdeg, (rdeg.shape[0], DEGW))


def _tc_mid(pflat, r1, b1, w, n_acc, blk):
    dh = r1.shape[1]
    do2 = w.shape[1]
    do = do2 // 2
    nblk = n_acc // blk
    grid = (nblk,)
    return pl.pallas_call(
        _mid_body,
        grid=grid,
        in_specs=[pl.BlockSpec((blk, dh + DEGW), lambda i: (i, 0)),
                  pl.BlockSpec((blk, dh + DEGW), lambda i: (i + nblk, 0)),
                  pl.BlockSpec((blk, dh), lambda i: (i, 0)),
                  pl.BlockSpec((1, dh), lambda i: (0, 0)),
                  pl.BlockSpec((dh, do2), lambda i: (0, 0))],
        out_specs=[pl.BlockSpec((blk, do), lambda i: (i, 0)),
                   pl.BlockSpec((blk, do), lambda i: (i, 0)),
                   pl.BlockSpec((blk, DEGW), lambda i: (i, 0))],
        out_shape=[jax.ShapeDtypeStruct((n_acc, do), jnp.float32),
                   jax.ShapeDtypeStruct((n_acc, do), jnp.float32),
                   jax.ShapeDtypeStruct((n_acc, DEGW), jnp.float32)],
    )(pflat, pflat, r1, b1, w)


def _fin_body(q0_ref, q1_ref, rd_ref, r2_ref, b2_ref, o_ref):
    rdeg = rd_ref[:, :1]
    o_ref[...] = (q0_ref[...] + q1_ref[...]) * rdeg + b2_ref[...] + r2_ref[...]


def _tc_fin(qflat, rd, r2, b2, n, n_acc, blk):
    do = r2.shape[1]
    nblk = n_acc // blk
    grid = (nblk,)
    return pl.pallas_call(
        _fin_body,
        grid=grid,
        in_specs=[pl.BlockSpec((blk, do), lambda i: (i, 0)),
                  pl.BlockSpec((blk, do), lambda i: (i + nblk, 0)),
                  pl.BlockSpec((blk, DEGW), lambda i: (i, 0)),
                  pl.BlockSpec((blk, do), lambda i: (i, 0)),
                  pl.BlockSpec((1, do), lambda i: (0, 0))],
        out_specs=pl.BlockSpec((blk, do), lambda i: (i, 0)),
        out_shape=jax.ShapeDtypeStruct((n, do), jnp.float32),
    )(qflat, qflat, rd, r2, b2)


def _sc_aggregate(table, ei_flat, dstr, zeros_d, n_acc, nb0, nb1):
    d = table.shape[1]
    rpt = n_acc // NS
    nzc = rpt // B
    mesh = plsc.VectorSubcoreMesh(core_axis_name="c", subcore_axis_name="s")

    def body(tab, ei, dr, zd_h, pout, src_v, dst_v, rows0, rows1, rows2,
             acc_sh, gsem, ssem):
        c = lax.axis_index("c")
        s = lax.axis_index("s")
        wid = c * NS + s
        stripe = s * rpt

        pltpu.sync_copy(dr.at[wid], dst_v)

        pltpu.sync_copy(zd_h, rows0)
        for i in range(nzc):
            pltpu.sync_copy(rows0, acc_sh.at[pl.ds(stripe + i * B, B)])
        plsc.subcore_barrier()

        bufs = (rows0, rows1, rows2)

        def run_pipe(nbat, base):
            pltpu.sync_copy(ei.at[pl.ds(base, nbat * B)],
                            src_v.at[pl.ds(0, nbat * B)])

            def src_at(t):
                return src_v.at[pl.ds(t * B, B)]

            gd = pltpu.async_copy(tab.at[src_at(0)], bufs[0], gsem)
            sd = None
            for t in range(nbat):
                gd.wait()
                if t + 1 < nbat:
                    gd = pltpu.async_copy(tab.at[src_at(t + 1)],
                                          bufs[(t + 1) % 3], gsem)
                if sd is not None:
                    sd.wait()
                sd = pltpu.async_copy(bufs[t % 3], acc_sh.at[dst_v.at[t]],
                                      ssem, add=True)
            sd.wait()

        if nb0 == nb1:
            run_pipe(nb0, wid * (nb0 * B))
        else:
            @pl.when(c == 0)
            def _():
                run_pipe(nb0, s * (nb0 * B))

            @pl.when(c == 1)
            def _():
                run_pipe(nb1, NS * (nb0 * B) + s * (nb1 * B))

        plsc.subcore_barrier()

        ods = {}
        for i in range(nzc):
            buf = bufs[i % 2]
            if i - 2 in ods:
                ods[i - 2].wait()
            pltpu.sync_copy(acc_sh.at[pl.ds(stripe + i * B, B)], buf)
            ods[i] = pltpu.async_copy(
                buf, pout.at[pl.ds(c * n_acc + stripe + i * B, B)], gsem)
        ods[nzc - 2].wait()
        ods[nzc - 1].wait()

    fn = pl.kernel(
        body,
        out_type=[jax.ShapeDtypeStruct((NC * n_acc, d), jnp.float32)],
        mesh=mesh,
        scratch_types=[
            pltpu.VMEM((nb0 * B,), jnp.int32),
            pltpu.VMEM((nb0, B), jnp.int32),
            pltpu.VMEM((B, d), jnp.float32),
            pltpu.VMEM((B, d), jnp.float32),
            pltpu.VMEM((B, d), jnp.float32),
            pltpu.VMEM_SHARED((n_acc, d), jnp.float32),
            pltpu.SemaphoreType.DMA,
            pltpu.SemaphoreType.DMA,
        ],
        compiler_params=pltpu.CompilerParams(use_tc_tiling_on_sc=False),
    )
    return fn(table, ei_flat, dstr, zeros_d)[0]


def _split_pack(vec, fill, nb0, nb1):
    e0 = NS * nb0 * B
    cap1 = NS * nb1 * B
    part0 = vec[:e0].reshape(NS, nb0, B)
    tail = vec[e0:]
    tail = jnp.concatenate([tail, fill[:cap1 - tail.shape[0]]])
    part1 = tail.reshape(NS, nb1, B)
    if nb1 < nb0:
        part1 = jnp.concatenate(
            [part1, jnp.zeros((NS, nb0 - nb1, B), jnp.int32)], axis=1)
    return jnp.concatenate([part0, part1], axis=0)


FRAC_L1 = 0.65
FRAC_L2 = 0.56


def kernel(x, edge_index, W_l1, b_l1, W_r1, W_l2, b_l2, W_r2):
    n, d_in = x.shape
    d_hid = W_l1.shape[1]
    d_out = W_l2.shape[1]
    e = edge_index.shape[1]

    blk = 1024
    n_acc = ((n + (NS * B) - 1) // (NS * B)) * (NS * B)
    tb = -(-e // (NS * B))
    nb0_1 = max(1, min(tb - 1, int(round(tb * FRAC_L1))))
    nb1_1 = tb - nb0_1
    nb0_2 = max(1, min(tb - 1, int(round(tb * FRAC_L2))))
    nb1_2 = tb - nb0_2

    ei_flat = edge_index.astype(jnp.int32).reshape(2 * e)
    dst = ei_flat[e:]
    spare = n_acc - n
    dst_fill = n + (jnp.arange(NS * B, dtype=jnp.int32) % spare)
    dstr1 = _split_pack(dst, dst_fill, nb0_1, nb1_1)
    dstr2 = _split_pack(dst, dst_fill, nb0_2, nb1_2)

    zeros_e = jnp.zeros((B, d_hid + DEGW), jnp.float32)
    zeros_o = jnp.zeros((B, d_out), jnp.float32)

    wcat1 = jnp.concatenate([W_l1, W_r1], axis=1)
    y1e, r1 = _tc_l1(x, wcat1, n_acc, blk)

    pflat = _sc_aggregate(y1e, ei_flat, dstr1, zeros_e, n_acc, nb0_1, nb1_1)

    wcat2 = jnp.concatenate([W_l2, W_r2], axis=1)
    y2, r2, rd = _tc_mid(pflat, r1, b_l1.reshape(1, d_hid), wcat2,
                         n_acc, blk)

    qflat = _sc_aggregate(y2, ei_flat, dstr2, zeros_o, n_acc, nb0_2, nb1_2)

    return _tc_fin(qflat, rd, r2, b_l2.reshape(1, d_out), n, n_acc, blk)


# speedup vs baseline: 2.3543x; 1.1044x over previous
import jax
import jax.numpy as jnp
from jax import lax
from jax.experimental import pallas as pl
from jax.experimental.pallas import tpu as pltpu
from jax.experimental.pallas import tpu_sc as plsc

NC = 2
NS = 16
NW = NC * NS
B = 128
DEGW = 16


def _l1_body(x_ref, w_ref, y1e_ref, r1_ref):
    yr = jnp.dot(x_ref[...], w_ref[...], preferred_element_type=jnp.float32)
    dh = w_ref.shape[1] // 2
    y1e_ref[:, :dh] = yr[:, :dh]
    y1e_ref[:, dh:] = jnp.ones((yr.shape[0], DEGW), jnp.float32)
    r1_ref[...] = yr[:, dh:]


def _tc_l1(x, w, n_acc, blk):
    d = x.shape[1]
    dh = w.shape[1] // 2
    grid = (n_acc // blk,)
    return pl.pallas_call(
        _l1_body,
        grid=grid,
        in_specs=[pl.BlockSpec((blk, d), lambda i: (i, 0)),
                  pl.BlockSpec((d, 2 * dh), lambda i: (0, 0))],
        out_specs=[pl.BlockSpec((blk, dh + DEGW), lambda i: (i, 0)),
                   pl.BlockSpec((blk, dh), lambda i: (i, 0))],
        out_shape=[jax.ShapeDtypeStruct((n_acc, dh + DEGW), jnp.float32),
                   jax.ShapeDtypeStruct((n_acc, dh), jnp.float32)],
    )(x, w)


def _mid_body(p0_ref, p1_ref, r1_ref, b1_ref, w_ref, y2_ref, r2_ref, rd_ref):
    dh = r1_ref.shape[1]
    deg = p0_ref[:, dh:dh + 1] + p1_ref[:, dh:dh + 1]
    rdeg = 1.0 / jnp.maximum(deg, 1.0)
    h = (p0_ref[:, :dh] + p1_ref[:, :dh]) * rdeg + b1_ref[...] + r1_ref[...]
    h = jnp.maximum(h, 0.0)
    yr = jnp.dot(h, w_ref[...], preferred_element_type=jnp.float32)
    do = w_ref.shape[1] // 2
    y2_ref[...] = yr[:, :do]
    r2_ref[...] = yr[:, do:]
    rd_ref[...] = jnp.broadcast_to(rdeg, (rdeg.shape[0], DEGW))


def _tc_mid(pflat, r1, b1, w, n_acc, blk):
    dh = r1.shape[1]
    do2 = w.shape[1]
    do = do2 // 2
    nblk = n_acc // blk
    grid = (nblk,)
    return pl.pallas_call(
        _mid_body,
        grid=grid,
        in_specs=[pl.BlockSpec((blk, dh + DEGW), lambda i: (i, 0)),
                  pl.BlockSpec((blk, dh + DEGW), lambda i: (i + nblk, 0)),
                  pl.BlockSpec((blk, dh), lambda i: (i, 0)),
                  pl.BlockSpec((1, dh), lambda i: (0, 0)),
                  pl.BlockSpec((dh, do2), lambda i: (0, 0))],
        out_specs=[pl.BlockSpec((blk, do), lambda i: (i, 0)),
                   pl.BlockSpec((blk, do), lambda i: (i, 0)),
                   pl.BlockSpec((blk, DEGW), lambda i: (i, 0))],
        out_shape=[jax.ShapeDtypeStruct((n_acc, do), jnp.float32),
                   jax.ShapeDtypeStruct((n_acc, do), jnp.float32),
                   jax.ShapeDtypeStruct((n_acc, DEGW), jnp.float32)],
    )(pflat, pflat, r1, b1, w)


def _fin_body(q0_ref, q1_ref, rd_ref, r2_ref, b2_ref, o_ref):
    rdeg = rd_ref[:, :1]
    o_ref[...] = (q0_ref[...] + q1_ref[...]) * rdeg + b2_ref[...] + r2_ref[...]


def _tc_fin(qflat, rd, r2, b2, n, n_acc, blk):
    do = r2.shape[1]
    nblk = n_acc // blk
    grid = (nblk,)
    return pl.pallas_call(
        _fin_body,
        grid=grid,
        in_specs=[pl.BlockSpec((blk, do), lambda i: (i, 0)),
                  pl.BlockSpec((blk, do), lambda i: (i + nblk, 0)),
                  pl.BlockSpec((blk, DEGW), lambda i: (i, 0)),
                  pl.BlockSpec((blk, do), lambda i: (i, 0)),
                  pl.BlockSpec((1, do), lambda i: (0, 0))],
        out_specs=pl.BlockSpec((blk, do), lambda i: (i, 0)),
        out_shape=jax.ShapeDtypeStruct((n, do), jnp.float32),
    )(qflat, qflat, rd, r2, b2)


def _sc_aggregate(table, ei_flat, dstr, zeros_d, n_acc, nb0, nb1):
    d = table.shape[1]
    rpt = n_acc // NS
    nzc = rpt // B
    mesh = plsc.VectorSubcoreMesh(core_axis_name="c", subcore_axis_name="s")

    def body(tab, ei, dr, zd_h, pout, src_v, dst_v, rows0, rows1, rows2,
             acc_sh, gsem, ssem):
        c = lax.axis_index("c")
        s = lax.axis_index("s")
        wid = c * NS + s
        stripe = s * rpt

        pltpu.sync_copy(dr.at[wid], dst_v)

        pltpu.sync_copy(zd_h, rows0)
        for i in range(nzc):
            pltpu.sync_copy(rows0, acc_sh.at[pl.ds(stripe + i * B, B)])
        plsc.subcore_barrier()

        bufs = (rows0, rows1, rows2)

        def run_pipe(nbat, base):
            pltpu.sync_copy(ei.at[pl.ds(base, nbat * B)],
                            src_v.at[pl.ds(0, nbat * B)])

            def src_at(t):
                return src_v.at[pl.ds(t * B, B)]

            gd = pltpu.async_copy(tab.at[src_at(0)], bufs[0], gsem)
            sd = None
            for t in range(nbat):
                gd.wait()
                if t + 1 < nbat:
                    gd = pltpu.async_copy(tab.at[src_at(t + 1)],
                                          bufs[(t + 1) % 3], gsem)
                if sd is not None:
                    sd.wait()
                sd = pltpu.async_copy(bufs[t % 3], acc_sh.at[dst_v.at[t]],
                                      ssem, add=True)
            sd.wait()

        if nb0 == nb1:
            run_pipe(nb0, wid * (nb0 * B))
        else:
            @pl.when(c == 0)
            def _():
                run_pipe(nb0, s * (nb0 * B))

            @pl.when(c == 1)
            def _():
                run_pipe(nb1, NS * (nb0 * B) + s * (nb1 * B))

        plsc.subcore_barrier()

        ods = {}
        for i in range(nzc):
            buf = bufs[i % 2]
            if i - 2 in ods:
                ods[i - 2].wait()
            pltpu.sync_copy(acc_sh.at[pl.ds(stripe + i * B, B)], buf)
            ods[i] = pltpu.async_copy(
                buf, pout.at[pl.ds(c * n_acc + stripe + i * B, B)], gsem)
        ods[nzc - 2].wait()
        ods[nzc - 1].wait()

    fn = pl.kernel(
        body,
        out_type=[jax.ShapeDtypeStruct((NC * n_acc, d), jnp.float32)],
        mesh=mesh,
        scratch_types=[
            pltpu.VMEM((nb0 * B,), jnp.int32),
            pltpu.VMEM((nb0, B), jnp.int32),
            pltpu.VMEM((B, d), jnp.float32),
            pltpu.VMEM((B, d), jnp.float32),
            pltpu.VMEM((B, d), jnp.float32),
            pltpu.VMEM_SHARED((n_acc, d), jnp.float32),
            pltpu.SemaphoreType.DMA,
            pltpu.SemaphoreType.DMA,
        ],
        compiler_params=pltpu.CompilerParams(use_tc_tiling_on_sc=False),
    )
    return fn(table, ei_flat, dstr, zeros_d)[0]


def _split_pack(vec, fill, nb0, nb1):
    e0 = NS * nb0 * B
    cap1 = NS * nb1 * B
    part0 = vec[:e0].reshape(NS, nb0, B)
    tail = vec[e0:]
    tail = jnp.concatenate([tail, fill[:cap1 - tail.shape[0]]])
    part1 = tail.reshape(NS, nb1, B)
    if nb1 < nb0:
        part1 = jnp.concatenate(
            [part1, jnp.zeros((NS, nb0 - nb1, B), jnp.int32)], axis=1)
    return jnp.concatenate([part0, part1], axis=0)


FRAC_L1 = 0.52
FRAC_L2 = 0.505


def kernel(x, edge_index, W_l1, b_l1, W_r1, W_l2, b_l2, W_r2):
    n, d_in = x.shape
    d_hid = W_l1.shape[1]
    d_out = W_l2.shape[1]
    e = edge_index.shape[1]

    blk = 1024
    n_acc = ((n + (NS * B) - 1) // (NS * B)) * (NS * B)
    tb = -(-e // (NS * B))
    nb0_1 = max(1, min(tb - 1, int(round(tb * FRAC_L1))))
    nb1_1 = tb - nb0_1
    nb0_2 = max(1, min(tb - 1, int(round(tb * FRAC_L2))))
    nb1_2 = tb - nb0_2

    ei_flat = edge_index.astype(jnp.int32).reshape(2 * e)
    dst = ei_flat[e:]
    spare = n_acc - n
    dst_fill = n + (jnp.arange(NS * B, dtype=jnp.int32) % spare)
    dstr1 = _split_pack(dst, dst_fill, nb0_1, nb1_1)
    dstr2 = _split_pack(dst, dst_fill, nb0_2, nb1_2)

    zeros_e = jnp.zeros((B, d_hid + DEGW), jnp.float32)
    zeros_o = jnp.zeros((B, d_out), jnp.float32)

    wcat1 = jnp.concatenate([W_l1, W_r1], axis=1)
    y1e, r1 = _tc_l1(x, wcat1, n_acc, blk)

    pflat = _sc_aggregate(y1e, ei_flat, dstr1, zeros_e, n_acc, nb0_1, nb1_1)

    wcat2 = jnp.concatenate([W_l2, W_r2], axis=1)
    y2, r2, rd = _tc_mid(pflat, r1, b_l1.reshape(1, d_hid), wcat2,
                         n_acc, blk)

    qflat = _sc_aggregate(y2, ei_flat, dstr2, zeros_o, n_acc, nb0_2, nb1_2)

    return _tc_fin(qflat, rd, r2, b_l2.reshape(1, d_out), n, n_acc, blk)


# speedup vs baseline: 2.3851x; 1.0131x over previous
import jax
import jax.numpy as jnp
from jax import lax
from jax.experimental import pallas as pl
from jax.experimental.pallas import tpu as pltpu
from jax.experimental.pallas import tpu_sc as plsc

NC = 2
NS = 16
NW = NC * NS
B = 128
DEGW = 16


def _l1_body(x_ref, w_ref, y1e_ref, r1_ref):
    yr = jnp.dot(x_ref[...], w_ref[...], preferred_element_type=jnp.float32)
    dh = w_ref.shape[1] // 2
    y1e_ref[:, :dh] = yr[:, :dh]
    y1e_ref[:, dh:] = jnp.ones((yr.shape[0], DEGW), jnp.float32)
    r1_ref[...] = yr[:, dh:]


def _tc_l1(x, w, n_acc, blk):
    d = x.shape[1]
    dh = w.shape[1] // 2
    grid = (n_acc // blk,)
    return pl.pallas_call(
        _l1_body,
        grid=grid,
        in_specs=[pl.BlockSpec((blk, d), lambda i: (i, 0)),
                  pl.BlockSpec((d, 2 * dh), lambda i: (0, 0))],
        out_specs=[pl.BlockSpec((blk, dh + DEGW), lambda i: (i, 0)),
                   pl.BlockSpec((blk, dh), lambda i: (i, 0))],
        out_shape=[jax.ShapeDtypeStruct((n_acc, dh + DEGW), jnp.float32),
                   jax.ShapeDtypeStruct((n_acc, dh), jnp.float32)],
    )(x, w)


def _mid_body(p0_ref, p1_ref, r1_ref, b1_ref, w_ref, y2_ref, r2_ref, rd_ref):
    dh = r1_ref.shape[1]
    deg = p0_ref[:, dh:dh + 1] + p1_ref[:, dh:dh + 1]
    rdeg = 1.0 / jnp.maximum(deg, 1.0)
    h = (p0_ref[:, :dh] + p1_ref[:, :dh]) * rdeg + b1_ref[...] + r1_ref[...]
    h = jnp.maximum(h, 0.0)
    yr = jnp.dot(h, w_ref[...], preferred_element_type=jnp.float32)
    do = w_ref.shape[1] // 2
    y2_ref[...] = yr[:, :do]
    r2_ref[...] = yr[:, do:]
    rd_ref[...] = jnp.broadcast_to(rdeg, (rdeg.shape[0], DEGW))


def _tc_mid(pflat, r1, b1, w, n_acc, blk):
    dh = r1.shape[1]
    do2 = w.shape[1]
    do = do2 // 2
    nblk = n_acc // blk
    grid = (nblk,)
    return pl.pallas_call(
        _mid_body,
        grid=grid,
        in_specs=[pl.BlockSpec((blk, dh + DEGW), lambda i: (i, 0)),
                  pl.BlockSpec((blk, dh + DEGW), lambda i: (i + nblk, 0)),
                  pl.BlockSpec((blk, dh), lambda i: (i, 0)),
                  pl.BlockSpec((1, dh), lambda i: (0, 0)),
                  pl.BlockSpec((dh, do2), lambda i: (0, 0))],
        out_specs=[pl.BlockSpec((blk, do), lambda i: (i, 0)),
                   pl.BlockSpec((blk, do), lambda i: (i, 0)),
                   pl.BlockSpec((blk, DEGW), lambda i: (i, 0))],
        out_shape=[jax.ShapeDtypeStruct((n_acc, do), jnp.float32),
                   jax.ShapeDtypeStruct((n_acc, do), jnp.float32),
                   jax.ShapeDtypeStruct((n_acc, DEGW), jnp.float32)],
    )(pflat, pflat, r1, b1, w)


def _fin_body(q0_ref, q1_ref, rd_ref, r2_ref, b2_ref, o_ref):
    rdeg = rd_ref[:, :1]
    o_ref[...] = (q0_ref[...] + q1_ref[...]) * rdeg + b2_ref[...] + r2_ref[...]


def _tc_fin(qflat, rd, r2, b2, n, n_acc, blk):
    do = r2.shape[1]
    nblk = n_acc // blk
    grid = (nblk,)
    return pl.pallas_call(
        _fin_body,
        grid=grid,
        in_specs=[pl.BlockSpec((blk, do), lambda i: (i, 0)),
                  pl.BlockSpec((blk, do), lambda i: (i + nblk, 0)),
                  pl.BlockSpec((blk, DEGW), lambda i: (i, 0)),
                  pl.BlockSpec((blk, do), lambda i: (i, 0)),
                  pl.BlockSpec((1, do), lambda i: (0, 0))],
        out_specs=pl.BlockSpec((blk, do), lambda i: (i, 0)),
        out_shape=jax.ShapeDtypeStruct((n, do), jnp.float32),
    )(qflat, qflat, rd, r2, b2)


def _sc_aggregate(table, ei_flat, dstr, zeros_d, n_acc, nb0, nb1):
    d = table.shape[1]
    rpt = n_acc // NS
    nzc = rpt // B
    mesh = plsc.VectorSubcoreMesh(core_axis_name="c", subcore_axis_name="s")

    def body(tab, ei, dr, zd_h, pout, src_v, dst_v, rows0, rows1, rows2,
             acc_sh, gsem, ssem):
        c = lax.axis_index("c")
        s = lax.axis_index("s")
        wid = c * NS + s
        stripe = s * rpt

        pltpu.sync_copy(dr.at[wid], dst_v)

        pltpu.sync_copy(zd_h, rows0)
        for i in range(nzc):
            pltpu.sync_copy(rows0, acc_sh.at[pl.ds(stripe + i * B, B)])
        plsc.subcore_barrier()

        bufs = (rows0, rows1, rows2)

        def run_pipe(nbat, base):
            pltpu.sync_copy(ei.at[pl.ds(base, nbat * B)],
                            src_v.at[pl.ds(0, nbat * B)])

            def src_at(t):
                return src_v.at[pl.ds(t * B, B)]

            gd = pltpu.async_copy(tab.at[src_at(0)], bufs[0], gsem)
            sd = None
            for t in range(nbat):
                gd.wait()
                if t + 1 < nbat:
                    gd = pltpu.async_copy(tab.at[src_at(t + 1)],
                                          bufs[(t + 1) % 3], gsem)
                if sd is not None:
                    sd.wait()
                sd = pltpu.async_copy(bufs[t % 3], acc_sh.at[dst_v.at[t]],
                                      ssem, add=True)
            sd.wait()

        if nb0 == nb1:
            run_pipe(nb0, wid * (nb0 * B))
        else:
            @pl.when(c == 0)
            def _():
                run_pipe(nb0, s * (nb0 * B))

            @pl.when(c == 1)
            def _():
                run_pipe(nb1, NS * (nb0 * B) + s * (nb1 * B))

        plsc.subcore_barrier()

        ods = {}
        for i in range(nzc):
            buf = bufs[i % 2]
            if i - 2 in ods:
                ods[i - 2].wait()
            pltpu.sync_copy(acc_sh.at[pl.ds(stripe + i * B, B)], buf)
            ods[i] = pltpu.async_copy(
                buf, pout.at[pl.ds(c * n_acc + stripe + i * B, B)], gsem)
        ods[nzc - 2].wait()
        ods[nzc - 1].wait()

    fn = pl.kernel(
        body,
        out_type=[jax.ShapeDtypeStruct((NC * n_acc, d), jnp.float32)],
        mesh=mesh,
        scratch_types=[
            pltpu.VMEM((nb0 * B,), jnp.int32),
            pltpu.VMEM((nb0, B), jnp.int32),
            pltpu.VMEM((B, d), jnp.float32),
            pltpu.VMEM((B, d), jnp.float32),
            pltpu.VMEM((B, d), jnp.float32),
            pltpu.VMEM_SHARED((n_acc, d), jnp.float32),
            pltpu.SemaphoreType.DMA,
            pltpu.SemaphoreType.DMA,
        ],
        compiler_params=pltpu.CompilerParams(use_tc_tiling_on_sc=False),
    )
    return fn(table, ei_flat, dstr, zeros_d)[0]


def _split_pack(vec, fill, nb0, nb1):
    e0 = NS * nb0 * B
    cap1 = NS * nb1 * B
    part0 = vec[:e0].reshape(NS, nb0, B)
    tail = vec[e0:]
    tail = jnp.concatenate([tail, fill[:cap1 - tail.shape[0]]])
    part1 = tail.reshape(NS, nb1, B)
    if nb1 < nb0:
        part1 = jnp.concatenate(
            [part1, jnp.zeros((NS, nb0 - nb1, B), jnp.int32)], axis=1)
    return jnp.concatenate([part0, part1], axis=0)


FRAC_L1 = 0.505
FRAC_L2 = 0.505


def kernel(x, edge_index, W_l1, b_l1, W_r1, W_l2, b_l2, W_r2):
    n, d_in = x.shape
    d_hid = W_l1.shape[1]
    d_out = W_l2.shape[1]
    e = edge_index.shape[1]

    blk = 1024
    n_acc = ((n + (NS * B) - 1) // (NS * B)) * (NS * B)
    tb = -(-e // (NS * B))
    nb0_1 = max(1, min(tb - 1, int(round(tb * FRAC_L1))))
    nb1_1 = tb - nb0_1
    nb0_2 = max(1, min(tb - 1, int(round(tb * FRAC_L2))))
    nb1_2 = tb - nb0_2

    ei_flat = edge_index.astype(jnp.int32).reshape(2 * e)
    dst = ei_flat[e:]
    spare = n_acc - n
    dst_fill = n + (jnp.arange(NS * B, dtype=jnp.int32) % spare)
    dstr1 = _split_pack(dst, dst_fill, nb0_1, nb1_1)
    dstr2 = _split_pack(dst, dst_fill, nb0_2, nb1_2)

    zeros_e = jnp.zeros((B, d_hid + DEGW), jnp.float32)
    zeros_o = jnp.zeros((B, d_out), jnp.float32)

    wcat1 = jnp.concatenate([W_l1, W_r1], axis=1)
    y1e, r1 = _tc_l1(x, wcat1, n_acc, blk)

    pflat = _sc_aggregate(y1e, ei_flat, dstr1, zeros_e, n_acc, nb0_1, nb1_1)

    wcat2 = jnp.concatenate([W_l2, W_r2], axis=1)
    y2, r2, rd = _tc_mid(pflat, r1, b_l1.reshape(1, d_hid), wcat2,
                         n_acc, blk)

    qflat = _sc_aggregate(y2, ei_flat, dstr2, zeros_o, n_acc, nb0_2, nb1_2)

    return _tc_fin(qflat, rd, r2, b_l2.reshape(1, d_out), n, n_acc, blk)
